# Initial kernel scaffold; baseline (speedup 1.0000x reference)
import jax
import jax.numpy as jnp
from jax.experimental import pallas as pl


def kernel(x, w9_1, b_1, w9_2, b_2, w9_3, b_3, w9_4, b_4, w9_5, b_5, w9_6, b_6, w1, b1, wh, bh, s1, s2, s3):
    raise NotImplementedError("write your pallas kernel here")


# speedup vs baseline: 2.2105x; 2.2105x over previous
import functools

import jax
import jax.numpy as jnp
from jax import lax
from jax.experimental import pallas as pl
from jax.experimental.pallas import tpu as pltpu

_SZ = 32
_BIMG = 16


def _conv3x3_relu(a, wcat, bias, hw, w):
    m, c = a.shape
    cout = wcat.shape[1] // 3
    zero = jnp.zeros((), a.dtype)

    pin = lax.broadcasted_iota(jnp.int32, (m, c), 0) & (hw - 1)
    up = jnp.where(pin >= w, jnp.pad(a, ((w, 0), (0, 0)))[:m], zero)
    dn = jnp.where(pin < hw - w, jnp.pad(a, ((0, w), (0, 0)))[w:], zero)
    cy = jnp.concatenate([up, a, dn], axis=1)

    z = jnp.dot(cy, wcat, preferred_element_type=jnp.float32)

    xo = lax.broadcasted_iota(jnp.int32, (m, cout), 0) & (w - 1)
    zf = jnp.zeros((), jnp.float32)
    left = jnp.where(xo != 0, jnp.pad(z[:, :cout], ((1, 0), (0, 0)))[:m], zf)
    right = jnp.where(xo != w - 1,
                      jnp.pad(z[:, 2 * cout:], ((0, 1), (0, 0)))[1:], zf)
    y = z[:, cout:2 * cout] + left + right + bias
    return jnp.maximum(y, 0.0)


def _pool2x2(a, w):
    m, c = a.shape
    v = a.reshape(m // (2 * w), 2, w, c)
    t = jnp.maximum(v[:, 0], v[:, 1])
    v2 = t.reshape(m // 4, 2, c)
    return jnp.maximum(v2[:, 0], v2[:, 1])


def _tower_kernel(x_ref, wc1, bb1, wc2, bb2, wc3, bb3, wc4, bb4,
                  wc5, bb5, wc6, bb6, o_ref, *, bimg, size):
    h1, h2, h3 = size, size // 2, size // 4
    a = x_ref[...].reshape(bimg * h1 * h1, 3)
    a = _conv3x3_relu(a, wc1[...], bb1[...], h1 * h1, h1).astype(jnp.bfloat16)
    a = _conv3x3_relu(a, wc2[...], bb2[...], h1 * h1, h1)
    a = _pool2x2(a, h1).astype(jnp.bfloat16)
    a = _conv3x3_relu(a, wc3[...], bb3[...], h2 * h2, h2).astype(jnp.bfloat16)
    a = _conv3x3_relu(a, wc4[...], bb4[...], h2 * h2, h2)
    a = _pool2x2(a, h2).astype(jnp.bfloat16)
    a = _conv3x3_relu(a, wc5[...], bb5[...], h3 * h3, h3).astype(jnp.bfloat16)
    a = _conv3x3_relu(a, wc6[...], bb6[...], h3 * h3, h3)
    a = _pool2x2(a, h3)
    o_ref[...] = a.astype(o_ref.dtype)


def _fc_kernel(f_ref, w1_ref, b1_ref, wh_ref, bh_ref, o_ref):
    h = jnp.dot(f_ref[...], w1_ref[...],
                preferred_element_type=jnp.float32) + b1_ref[...]
    h = jnp.maximum(h, 0.0).astype(jnp.bfloat16)
    o_ref[...] = jnp.dot(h, wh_ref[...],
                         preferred_element_type=jnp.float32) + bh_ref[...]


def _pack_conv(w9):
    cols = []
    for ox in range(3):
        cols.append(jnp.concatenate([w9[0 + ox], w9[3 + ox], w9[6 + ox]],
                                    axis=0))
    return jnp.concatenate(cols, axis=1).astype(jnp.bfloat16)


def kernel(x, w9_1, b_1, w9_2, b_2, w9_3, b_3, w9_4, b_4, w9_5, b_5,
           w9_6, b_6, w1, b1, wh, bh, s1, s2, s3):
    del s1, s2, s3
    n = x.shape[0]
    size = _SZ
    sf = size // 8
    ss = sf * sf
    bimg = _BIMG

    x_flat = jnp.transpose(x, (0, 2, 3, 1)).reshape(
        n, size * size, 3).astype(jnp.bfloat16)

    wcs = [_pack_conv(w) for w in (w9_1, w9_2, w9_3, w9_4, w9_5, w9_6)]
    bbs = [b_1, b_2, b_3, b_4, b_5, b_6]

    def const_spec(shape):
        zeros = (0,) * len(shape)
        return pl.BlockSpec(shape, lambda i, _z=zeros: _z)

    in_specs = [pl.BlockSpec((bimg, size * size, 3), lambda i: (i, 0, 0))]
    args = [x_flat]
    for wc, bb in zip(wcs, bbs):
        in_specs += [const_spec(wc.shape), const_spec(bb.shape)]
        args += [wc, bb]

    feat = pl.pallas_call(
        functools.partial(_tower_kernel, bimg=bimg, size=size),
        out_shape=jax.ShapeDtypeStruct((n * ss, 128), jnp.bfloat16),
        grid=(n // bimg,),
        in_specs=in_specs,
        out_specs=pl.BlockSpec((bimg * ss, 128), lambda i: (i, 0)),
        compiler_params=pltpu.CompilerParams(
            dimension_semantics=("parallel",),
            vmem_limit_bytes=100 * 1024 * 1024),
    )(*args)

    feat2 = feat.reshape(n, ss * 128)
    hid = w1.shape[1]
    npad = wh.shape[1]
    nblk = n // 2
    y_all = pl.pallas_call(
        _fc_kernel,
        out_shape=jax.ShapeDtypeStruct((n, npad), jnp.float32),
        grid=(2,),
        in_specs=[
            pl.BlockSpec((nblk, ss * 128), lambda i: (i, 0)),
            pl.BlockSpec((ss * 128, hid), lambda i: (0, 0)),
            pl.BlockSpec((1, hid), lambda i: (0, 0)),
            pl.BlockSpec((hid, npad), lambda i: (0, 0)),
            pl.BlockSpec((1, npad), lambda i: (0, 0)),
        ],
        out_specs=pl.BlockSpec((nblk, npad), lambda i: (i, 0)),
        compiler_params=pltpu.CompilerParams(
            dimension_semantics=("parallel",),
            vmem_limit_bytes=64 * 1024 * 1024),
    )(feat2, w1.astype(jnp.bfloat16), b1, wh.astype(jnp.bfloat16), bh)

    outs, off = [], 0
    for _ in range(10):
        outs.append(y_all[:, off:off + 10])
        off += 10
    return outs


# speedup vs baseline: 2.7085x; 1.2253x over previous
import functools

import jax
import jax.numpy as jnp
from jax import lax
from jax.experimental import pallas as pl
from jax.experimental.pallas import tpu as pltpu

_SZ = 32
_BIMG = 16


def _conv3x3_relu(a3, wcat, bias, w):
    b_, hw, c = a3.shape
    cout = wcat.shape[1] // 3
    zp = jnp.zeros((b_, w, c), a3.dtype)
    up = jnp.concatenate([zp, a3[:, :hw - w, :]], axis=1)
    dn = jnp.concatenate([a3[:, w:, :], zp], axis=1)
    cy = jnp.concatenate([up, a3, dn], axis=2)

    z = jnp.dot(cy.reshape(b_ * hw, 3 * c), wcat,
                preferred_element_type=jnp.float32)

    m = b_ * hw
    xo = lax.broadcasted_iota(jnp.int32, (m, cout), 0) & (w - 1)
    zf = jnp.zeros((), jnp.float32)
    left = jnp.where(xo != 0, jnp.pad(z[:, :cout], ((1, 0), (0, 0)))[:m], zf)
    right = jnp.where(xo != w - 1,
                      jnp.pad(z[:, 2 * cout:], ((0, 1), (0, 0)))[1:], zf)
    y = z[:, cout:2 * cout] + left + right + bias
    return jnp.maximum(y, 0.0).reshape(b_, hw, cout)


def _pool2x2(a3, w):
    b_, hw, c = a3.shape
    m = b_ * hw
    v = a3.reshape(m // (2 * w), 2, w, c)
    t = jnp.maximum(v[:, 0], v[:, 1])
    v2 = t.reshape(m // 4, 2, c)
    return jnp.maximum(v2[:, 0], v2[:, 1]).reshape(b_, hw // 4, c)


def _tower_kernel(x_ref, wc1, bb1, wc2, bb2, wc3, bb3, wc4, bb4,
                  wc5, bb5, wc6, bb6, o_ref, *, bimg, size):
    h1, h2, h3 = size, size // 2, size // 4
    a = jnp.transpose(x_ref[...], (0, 2, 1)).astype(jnp.bfloat16)
    a = _conv3x3_relu(a, wc1[...], bb1[...], h1).astype(jnp.bfloat16)
    a = _conv3x3_relu(a, wc2[...], bb2[...], h1)
    a = _pool2x2(a, h1).astype(jnp.bfloat16)
    a = _conv3x3_relu(a, wc3[...], bb3[...], h2).astype(jnp.bfloat16)
    a = _conv3x3_relu(a, wc4[...], bb4[...], h2)
    a = _pool2x2(a, h2).astype(jnp.bfloat16)
    a = _conv3x3_relu(a, wc5[...], bb5[...], h3).astype(jnp.bfloat16)
    a = _conv3x3_relu(a, wc6[...], bb6[...], h3)
    a = _pool2x2(a, h3)
    o_ref[...] = a.reshape(o_ref.shape).astype(o_ref.dtype)


def _fc_kernel(f_ref, w1_ref, b1_ref, wh_ref, bh_ref, o_ref):
    h = jnp.dot(f_ref[...], w1_ref[...],
                preferred_element_type=jnp.float32) + b1_ref[...]
    h = jnp.maximum(h, 0.0).astype(jnp.bfloat16)
    o_ref[...] = jnp.dot(h, wh_ref[...],
                         preferred_element_type=jnp.float32) + bh_ref[...]


def _pack_conv(w9):
    cols = []
    for ox in range(3):
        cols.append(jnp.concatenate([w9[0 + ox], w9[3 + ox], w9[6 + ox]],
                                    axis=0))
    return jnp.concatenate(cols, axis=1).astype(jnp.bfloat16)


def kernel(x, w9_1, b_1, w9_2, b_2, w9_3, b_3, w9_4, b_4, w9_5, b_5,
           w9_6, b_6, w1, b1, wh, bh, s1, s2, s3):
    del s1, s2, s3
    n = x.shape[0]
    size = _SZ
    sf = size // 8
    ss = sf * sf
    bimg = _BIMG

    x_flat = x.reshape(n, 3, size * size)

    wcs = [_pack_conv(w) for w in (w9_1, w9_2, w9_3, w9_4, w9_5, w9_6)]
    bbs = [b_1, b_2, b_3, b_4, b_5, b_6]

    def const_spec(shape):
        zeros = (0,) * len(shape)
        return pl.BlockSpec(shape, lambda i, _z=zeros: _z)

    in_specs = [pl.BlockSpec((bimg, 3, size * size), lambda i: (i, 0, 0))]
    args = [x_flat]
    for wc, bb in zip(wcs, bbs):
        in_specs += [const_spec(wc.shape), const_spec(bb.shape)]
        args += [wc, bb]

    feat = pl.pallas_call(
        functools.partial(_tower_kernel, bimg=bimg, size=size),
        out_shape=jax.ShapeDtypeStruct((n * ss, 128), jnp.bfloat16),
        grid=(n // bimg,),
        in_specs=in_specs,
        out_specs=pl.BlockSpec((bimg * ss, 128), lambda i: (i, 0)),
        compiler_params=pltpu.CompilerParams(
            dimension_semantics=("parallel",),
            vmem_limit_bytes=100 * 1024 * 1024),
    )(*args)

    feat2 = feat.reshape(n, ss * 128)
    hid = w1.shape[1]
    npad = wh.shape[1]
    nblk = n // 2
    y_all = pl.pallas_call(
        _fc_kernel,
        out_shape=jax.ShapeDtypeStruct((n, npad), jnp.float32),
        grid=(2,),
        in_specs=[
            pl.BlockSpec((nblk, ss * 128), lambda i: (i, 0)),
            pl.BlockSpec((ss * 128, hid), lambda i: (0, 0)),
            pl.BlockSpec((1, hid), lambda i: (0, 0)),
            pl.BlockSpec((hid, npad), lambda i: (0, 0)),
            pl.BlockSpec((1, npad), lambda i: (0, 0)),
        ],
        out_specs=pl.BlockSpec((nblk, npad), lambda i: (i, 0)),
        compiler_params=pltpu.CompilerParams(
            dimension_semantics=("parallel",),
            vmem_limit_bytes=64 * 1024 * 1024),
    )(feat2, w1.astype(jnp.bfloat16), b1, wh.astype(jnp.bfloat16), bh)

    outs, off = [], 0
    for _ in range(10):
        outs.append(y_all[:, off:off + 10])
        off += 10
    return outs


# speedup vs baseline: 2.9425x; 1.0864x over previous
import functools

import jax
import jax.numpy as jnp
from jax import lax
from jax.experimental import pallas as pl
from jax.experimental.pallas import tpu as pltpu

_SZ = 32
_BIMG = 16


_LANE = 128


def _conv3x3_relu(a3, wcat, bias, w):
    b_, hw, c = a3.shape
    zp = jnp.zeros((b_, w, c), a3.dtype)
    up = jnp.concatenate([zp, a3[:, :hw - w, :]], axis=1)
    dn = jnp.concatenate([a3[:, w:, :], zp], axis=1)
    cy = jnp.concatenate([up, a3, dn], axis=2)

    z = jnp.dot(cy.reshape(b_ * hw, 3 * c), wcat,
                preferred_element_type=jnp.float32).astype(jnp.bfloat16)

    m = b_ * hw
    xo = lax.broadcasted_iota(jnp.int32, (m, c), 0) & (w - 1)
    zb = jnp.zeros((), jnp.bfloat16)
    left = jnp.where(xo != 0, jnp.pad(z[:, :c], ((1, 0), (0, 0)))[:m], zb)
    right = jnp.where(xo != w - 1,
                      jnp.pad(z[:, 2 * c:], ((0, 1), (0, 0)))[1:], zb)
    y = z[:, c:2 * c] + left + right + bias
    return jnp.maximum(y, zb).reshape(b_, hw, c)


def _pool2x2(a3, w):
    b_, hw, c = a3.shape
    m = b_ * hw
    v = a3.reshape(m // (2 * w), 2, w, c)
    t = jnp.maximum(v[:, 0], v[:, 1])
    v2 = t.reshape(m // 4, 2, c)
    return jnp.maximum(v2[:, 0], v2[:, 1]).reshape(b_, hw // 4, c)


def _tower_kernel(x_ref, wc1, bb1, wc2, bb2, wc3, bb3, wc4, bb4,
                  wc5, bb5, wc6, bb6, o_ref, *, bimg, size):
    h1, h2, h3 = size, size // 2, size // 4
    a = jnp.transpose(x_ref[...], (0, 2, 1)).astype(jnp.bfloat16)
    a = jnp.pad(a, ((0, 0), (0, 0), (0, _LANE - a.shape[2])))
    a = _conv3x3_relu(a, wc1[...], bb1[...], h1).astype(jnp.bfloat16)
    a = _conv3x3_relu(a, wc2[...], bb2[...], h1)
    a = _pool2x2(a, h1).astype(jnp.bfloat16)
    a = _conv3x3_relu(a, wc3[...], bb3[...], h2).astype(jnp.bfloat16)
    a = _conv3x3_relu(a, wc4[...], bb4[...], h2)
    a = _pool2x2(a, h2).astype(jnp.bfloat16)
    a = _conv3x3_relu(a, wc5[...], bb5[...], h3).astype(jnp.bfloat16)
    a = _conv3x3_relu(a, wc6[...], bb6[...], h3)
    a = _pool2x2(a, h3)
    o_ref[...] = a.reshape(o_ref.shape).astype(o_ref.dtype)


def _fc_kernel(f_ref, w1_ref, b1_ref, wh_ref, bh_ref, o_ref):
    h = jnp.dot(f_ref[...], w1_ref[...],
                preferred_element_type=jnp.float32) + b1_ref[...]
    h = jnp.maximum(h, 0.0).astype(jnp.bfloat16)
    o_ref[...] = jnp.dot(h, wh_ref[...],
                         preferred_element_type=jnp.float32) + bh_ref[...]


def _pack_conv(w9):
    cin, cout = w9.shape[1], w9.shape[2]
    wc = jnp.zeros((3 * _LANE, 3 * _LANE), jnp.float32)
    for ky in range(3):
        for ox in range(3):
            wc = wc.at[ky * _LANE:ky * _LANE + cin,
                       ox * _LANE:ox * _LANE + cout].set(w9[ky * 3 + ox])
    return wc.astype(jnp.bfloat16)


def _pack_bias(b):
    return jnp.pad(b, ((0, 0), (0, _LANE - b.shape[1]))).astype(jnp.bfloat16)


def kernel(x, w9_1, b_1, w9_2, b_2, w9_3, b_3, w9_4, b_4, w9_5, b_5,
           w9_6, b_6, w1, b1, wh, bh, s1, s2, s3):
    del s1, s2, s3
    n = x.shape[0]
    size = _SZ
    sf = size // 8
    ss = sf * sf
    bimg = _BIMG

    x_flat = x.reshape(n, 3, size * size)

    wcs = [_pack_conv(w) for w in (w9_1, w9_2, w9_3, w9_4, w9_5, w9_6)]
    bbs = [_pack_bias(b) for b in (b_1, b_2, b_3, b_4, b_5, b_6)]

    def const_spec(shape):
        zeros = (0,) * len(shape)
        return pl.BlockSpec(shape, lambda i, _z=zeros: _z)

    in_specs = [pl.BlockSpec((bimg, 3, size * size), lambda i: (i, 0, 0))]
    args = [x_flat]
    for wc, bb in zip(wcs, bbs):
        in_specs += [const_spec(wc.shape), const_spec(bb.shape)]
        args += [wc, bb]

    feat = pl.pallas_call(
        functools.partial(_tower_kernel, bimg=bimg, size=size),
        out_shape=jax.ShapeDtypeStruct((n * ss, 128), jnp.bfloat16),
        grid=(n // bimg,),
        in_specs=in_specs,
        out_specs=pl.BlockSpec((bimg * ss, 128), lambda i: (i, 0)),
        compiler_params=pltpu.CompilerParams(
            dimension_semantics=("parallel",),
            vmem_limit_bytes=100 * 1024 * 1024),
    )(*args)

    feat2 = feat.reshape(n, ss * 128)
    hid = w1.shape[1]
    npad = wh.shape[1]
    nblk = n // 2
    y_all = pl.pallas_call(
        _fc_kernel,
        out_shape=jax.ShapeDtypeStruct((n, npad), jnp.float32),
        grid=(2,),
        in_specs=[
            pl.BlockSpec((nblk, ss * 128), lambda i: (i, 0)),
            pl.BlockSpec((ss * 128, hid), lambda i: (0, 0)),
            pl.BlockSpec((1, hid), lambda i: (0, 0)),
            pl.BlockSpec((hid, npad), lambda i: (0, 0)),
            pl.BlockSpec((1, npad), lambda i: (0, 0)),
        ],
        out_specs=pl.BlockSpec((nblk, npad), lambda i: (i, 0)),
        compiler_params=pltpu.CompilerParams(
            dimension_semantics=("parallel",),
            vmem_limit_bytes=64 * 1024 * 1024),
    )(feat2, w1.astype(jnp.bfloat16), b1, wh.astype(jnp.bfloat16), bh)

    outs, off = [], 0
    for _ in range(10):
        outs.append(y_all[:, off:off + 10])
        off += 10
    return outs


# speedup vs baseline: 6.5352x; 2.2210x over previous
import functools

import jax
import jax.numpy as jnp
from jax import lax
from jax.experimental import pallas as pl
from jax.experimental.pallas import tpu as pltpu

_SZ = 32
_BIMG = 16
_G = 4


def _conv3x3_relu(a3, wcat, bias, w):
    g, hw, kb = a3.shape
    nb = wcat.shape[1] // 3
    zp = jnp.zeros((g, w, kb), a3.dtype)
    up = jnp.concatenate([zp, a3[:, :hw - w, :]], axis=1)
    dn = jnp.concatenate([a3[:, w:, :], zp], axis=1)
    cy = jnp.concatenate([up, a3, dn], axis=2)

    z = jnp.dot(cy.reshape(g * hw, 3 * kb), wcat,
                preferred_element_type=jnp.float32).astype(jnp.bfloat16)

    m = g * hw
    xo = lax.broadcasted_iota(jnp.int32, (m, nb), 0) & (w - 1)
    zb = jnp.zeros((), jnp.bfloat16)
    left = jnp.where(xo != 0, jnp.pad(z[:, :nb], ((1, 0), (0, 0)))[:m], zb)
    right = jnp.where(xo != w - 1,
                      jnp.pad(z[:, 2 * nb:], ((0, 1), (0, 0)))[1:], zb)
    y = z[:, nb:2 * nb] + left + right + bias
    return jnp.maximum(y, zb).reshape(g, hw, nb)


def _pool2x2(a3, w):
    g, hw, c = a3.shape
    m = g * hw
    v = a3.reshape(m // (2 * w), 2, w, c)
    t = jnp.maximum(v[:, 0], v[:, 1])
    v2 = t.reshape(m // 4, 2, c)
    return jnp.maximum(v2[:, 0], v2[:, 1]).reshape(g, hw // 4, c)


def _tower_kernel(x_ref, wc1, bb1, wc2, bb2, wc3, bb3, wc4, bb4,
                  wc5, bb5, wc6, bb6, o_ref, *, bimg, size):
    h1, h2, h3 = size, size // 2, size // 4
    at = jnp.transpose(x_ref[...], (0, 2, 1)).astype(jnp.bfloat16)
    xg = at.reshape(_G, bimg // _G, h1 * h1, 3)
    a = jnp.concatenate([xg[:, i] for i in range(bimg // _G)], axis=2)
    a = jnp.pad(a, ((0, 0), (0, 0), (0, 128 - a.shape[2])))

    a = _conv3x3_relu(a, wc1[...], bb1[...], h1)
    a = _conv3x3_relu(a, wc2[...], bb2[...], h1)
    a = _pool2x2(a, h1)
    a = _conv3x3_relu(a, wc3[...], bb3[...], h2)
    a = _conv3x3_relu(a, wc4[...], bb4[...], h2)
    a = _pool2x2(a, h2)
    a = _conv3x3_relu(a, wc5[...], bb5[...], h3)
    a = _conv3x3_relu(a, wc6[...], bb6[...], h3)
    a = _pool2x2(a, h3)
    o_ref[...] = a.astype(o_ref.dtype)


def _fc_kernel(f_ref, w1_ref, b1_ref, wh_ref, bh_ref, o_ref):
    h = jnp.dot(f_ref[...], w1_ref[...],
                preferred_element_type=jnp.float32) + b1_ref[...]
    h = jnp.maximum(h, 0.0).astype(jnp.bfloat16)
    o_ref[...] = jnp.dot(h, wh_ref[...],
                         preferred_element_type=jnp.float32) + bh_ref[...]


def _pack_conv(w9, kb1=None):
    cin, cout = w9.shape[1], w9.shape[2]
    w9r = w9.reshape(3, 3, cin, cout)
    eye = jnp.eye(_G, dtype=w9.dtype)
    t = jnp.einsum("kxco,ij->kicxjo", w9r, eye)
    t = t.reshape(3, _G * cin, 3 * _G * cout)
    if kb1 is not None:
        t = jnp.pad(t, ((0, 0), (0, kb1 - _G * cin), (0, 0)))
    return t.reshape(-1, 3 * _G * cout).astype(jnp.bfloat16)


def _pack_bias(b):
    return jnp.tile(b, (1, _G)).astype(jnp.bfloat16)


def _block_diag4(wm):
    k, nn = wm.shape
    eye = jnp.eye(_G, dtype=wm.dtype)
    t = jnp.einsum("kn,ij->ikjn", wm, eye)
    return t.reshape(_G * k, _G * nn).astype(jnp.bfloat16)


def kernel(x, w9_1, b_1, w9_2, b_2, w9_3, b_3, w9_4, b_4, w9_5, b_5,
           w9_6, b_6, w1, b1, wh, bh, s1, s2, s3):
    del s1, s2, s3
    n = x.shape[0]
    size = _SZ
    sf = size // 8
    ss = sf * sf
    bimg = _BIMG

    x_flat = x.reshape(n, 3, size * size)

    wcs = [_pack_conv(w9_1, kb1=128)] + [
        _pack_conv(w) for w in (w9_2, w9_3, w9_4, w9_5, w9_6)]
    bbs = [_pack_bias(b) for b in (b_1, b_2, b_3, b_4, b_5, b_6)]

    def const_spec(shape):
        zeros = (0,) * len(shape)
        return pl.BlockSpec(shape, lambda i, _z=zeros: _z)

    in_specs = [pl.BlockSpec((bimg, 3, size * size), lambda i: (i, 0, 0))]
    args = [x_flat]
    for wc, bb in zip(wcs, bbs):
        in_specs += [const_spec(wc.shape), const_spec(bb.shape)]
        args += [wc, bb]

    feat = pl.pallas_call(
        functools.partial(_tower_kernel, bimg=bimg, size=size),
        out_shape=jax.ShapeDtypeStruct((n // _G, ss, _G * 128), jnp.bfloat16),
        grid=(n // bimg,),
        in_specs=in_specs,
        out_specs=pl.BlockSpec((bimg // _G, ss, _G * 128), lambda i: (i, 0, 0)),
        compiler_params=pltpu.CompilerParams(
            dimension_semantics=("parallel",),
            vmem_limit_bytes=100 * 1024 * 1024),
    )(*args)

    feat2 = feat.reshape(n // _G, ss * _G * 128)
    w1r = w1.reshape(ss, 128, w1.shape[1])
    eye = jnp.eye(_G, dtype=w1.dtype)
    w1p = jnp.einsum("pch,ij->picjh", w1r, eye).reshape(
        ss * _G * 128, _G * w1.shape[1]).astype(jnp.bfloat16)
    b1p = jnp.tile(b1, (1, _G))
    whp = _block_diag4(wh)
    bhp = jnp.tile(bh, (1, _G))

    hid = _G * w1.shape[1]
    npad = _G * wh.shape[1]
    nrow = n // _G
    nblk = nrow // 2
    y_all = pl.pallas_call(
        _fc_kernel,
        out_shape=jax.ShapeDtypeStruct((nrow, npad), jnp.float32),
        grid=(2,),
        in_specs=[
            pl.BlockSpec((nblk, ss * _G * 128), lambda i: (i, 0)),
            pl.BlockSpec((ss * _G * 128, hid), lambda i: (0, 0)),
            pl.BlockSpec((1, hid), lambda i: (0, 0)),
            pl.BlockSpec((hid, npad), lambda i: (0, 0)),
            pl.BlockSpec((1, npad), lambda i: (0, 0)),
        ],
        out_specs=pl.BlockSpec((nblk, npad), lambda i: (i, 0)),
        compiler_params=pltpu.CompilerParams(
            dimension_semantics=("parallel",),
            vmem_limit_bytes=64 * 1024 * 1024),
    )(feat2, w1p, b1p, whp, bhp)

    y2 = y_all.reshape(n, wh.shape[1])
    outs, off = [], 0
    for _ in range(10):
        outs.append(y2[:, off:off + 10])
        off += 10
    return outs


# speedup vs baseline: 6.7317x; 1.0301x over previous
import functools

import jax
import jax.numpy as jnp
from jax import lax
from jax.experimental import pallas as pl
from jax.experimental.pallas import tpu as pltpu

_SZ = 32
_BIMG = 32
_G = 4


def _conv3x3_relu(a3, wcat, bias, w):
    g, hw, kb = a3.shape
    nb = wcat.shape[1] // 3
    zp = jnp.zeros((g, w, kb), a3.dtype)
    up = jnp.concatenate([zp, a3[:, :hw - w, :]], axis=1)
    dn = jnp.concatenate([a3[:, w:, :], zp], axis=1)
    cy = jnp.concatenate([up, a3, dn], axis=2)

    z = jnp.dot(cy.reshape(g * hw, 3 * kb), wcat,
                preferred_element_type=jnp.float32).astype(jnp.bfloat16)

    m = g * hw
    xo = lax.broadcasted_iota(jnp.int32, (m, nb), 0) & (w - 1)
    zb = jnp.zeros((), jnp.bfloat16)
    left = jnp.where(xo != 0, jnp.pad(z[:, :nb], ((1, 0), (0, 0)))[:m], zb)
    right = jnp.where(xo != w - 1,
                      jnp.pad(z[:, 2 * nb:], ((0, 1), (0, 0)))[1:], zb)
    y = z[:, nb:2 * nb] + left + right + bias
    return jnp.maximum(y, zb).reshape(g, hw, nb)


def _pool2x2(a3, w):
    g, hw, c = a3.shape
    m = g * hw
    v = a3.reshape(m // (2 * w), 2, w, c)
    t = jnp.maximum(v[:, 0], v[:, 1])
    v2 = t.reshape(m // 4, 2, c)
    return jnp.maximum(v2[:, 0], v2[:, 1]).reshape(g, hw // 4, c)


def _tower_kernel(x_ref, wc1, bb1, wc2, bb2, wc3, bb3, wc4, bb4,
                  wc5, bb5, wc6, bb6, o_ref, *, bimg, size):
    h1, h2, h3 = size, size // 2, size // 4
    at = jnp.transpose(x_ref[...], (0, 2, 1)).astype(jnp.bfloat16)
    xg = at.reshape(bimg // _G, _G, h1 * h1, 3)
    a = jnp.concatenate([xg[:, i] for i in range(_G)], axis=2)
    a = jnp.pad(a, ((0, 0), (0, 0), (0, 128 - a.shape[2])))

    a = _conv3x3_relu(a, wc1[...], bb1[...], h1)
    a = _conv3x3_relu(a, wc2[...], bb2[...], h1)
    a = _pool2x2(a, h1)
    a = _conv3x3_relu(a, wc3[...], bb3[...], h2)
    a = _conv3x3_relu(a, wc4[...], bb4[...], h2)
    a = _pool2x2(a, h2)
    a = _conv3x3_relu(a, wc5[...], bb5[...], h3)
    a = _conv3x3_relu(a, wc6[...], bb6[...], h3)
    a = _pool2x2(a, h3)
    o_ref[...] = a.astype(o_ref.dtype)


def _fc_kernel(f_ref, w1_ref, b1_ref, wh_ref, bh_ref, o_ref):
    h = jnp.dot(f_ref[...], w1_ref[...],
                preferred_element_type=jnp.float32) + b1_ref[...]
    h = jnp.maximum(h, 0.0).astype(jnp.bfloat16)
    o_ref[...] = jnp.dot(h, wh_ref[...],
                         preferred_element_type=jnp.float32) + bh_ref[...]


def _pack_conv(w9, kb1=None):
    cin, cout = w9.shape[1], w9.shape[2]
    w9r = w9.reshape(3, 3, cin, cout)
    eye = jnp.eye(_G, dtype=w9.dtype)
    t = jnp.einsum("kxco,ij->kicxjo", w9r, eye)
    t = t.reshape(3, _G * cin, 3 * _G * cout)
    if kb1 is not None:
        t = jnp.pad(t, ((0, 0), (0, kb1 - _G * cin), (0, 0)))
    return t.reshape(-1, 3 * _G * cout).astype(jnp.bfloat16)


def _pack_bias(b):
    return jnp.tile(b, (1, _G)).astype(jnp.bfloat16)


def _block_diag4(wm):
    k, nn = wm.shape
    eye = jnp.eye(_G, dtype=wm.dtype)
    t = jnp.einsum("kn,ij->ikjn", wm, eye)
    return t.reshape(_G * k, _G * nn).astype(jnp.bfloat16)


def kernel(x, w9_1, b_1, w9_2, b_2, w9_3, b_3, w9_4, b_4, w9_5, b_5,
           w9_6, b_6, w1, b1, wh, bh, s1, s2, s3):
    del s1, s2, s3
    n = x.shape[0]
    size = _SZ
    sf = size // 8
    ss = sf * sf
    bimg = _BIMG

    x_flat = x.reshape(n, 3, size * size)

    wcs = [_pack_conv(w9_1, kb1=128)] + [
        _pack_conv(w) for w in (w9_2, w9_3, w9_4, w9_5, w9_6)]
    bbs = [_pack_bias(b) for b in (b_1, b_2, b_3, b_4, b_5, b_6)]

    def const_spec(shape):
        zeros = (0,) * len(shape)
        return pl.BlockSpec(shape, lambda i, _z=zeros: _z)

    in_specs = [pl.BlockSpec((bimg, 3, size * size), lambda i: (i, 0, 0))]
    args = [x_flat]
    for wc, bb in zip(wcs, bbs):
        in_specs += [const_spec(wc.shape), const_spec(bb.shape)]
        args += [wc, bb]

    feat = pl.pallas_call(
        functools.partial(_tower_kernel, bimg=bimg, size=size),
        out_shape=jax.ShapeDtypeStruct((n // _G, ss, _G * 128), jnp.bfloat16),
        grid=(n // bimg,),
        in_specs=in_specs,
        out_specs=pl.BlockSpec((bimg // _G, ss, _G * 128), lambda i: (i, 0, 0)),
        compiler_params=pltpu.CompilerParams(
            dimension_semantics=("parallel",),
            vmem_limit_bytes=100 * 1024 * 1024),
    )(*args)

    feat2 = feat.reshape(n // _G, ss * _G * 128)
    w1r = w1.reshape(ss, 128, w1.shape[1])
    eye = jnp.eye(_G, dtype=w1.dtype)
    w1p = jnp.einsum("pch,ij->picjh", w1r, eye).reshape(
        ss * _G * 128, _G * w1.shape[1]).astype(jnp.bfloat16)
    b1p = jnp.tile(b1, (1, _G))
    whp = _block_diag4(wh)
    bhp = jnp.tile(bh, (1, _G))

    hid = _G * w1.shape[1]
    npad = _G * wh.shape[1]
    nrow = n // _G
    nblk = nrow // 2
    y_all = pl.pallas_call(
        _fc_kernel,
        out_shape=jax.ShapeDtypeStruct((nrow, npad), jnp.float32),
        grid=(2,),
        in_specs=[
            pl.BlockSpec((nblk, ss * _G * 128), lambda i: (i, 0)),
            pl.BlockSpec((ss * _G * 128, hid), lambda i: (0, 0)),
            pl.BlockSpec((1, hid), lambda i: (0, 0)),
            pl.BlockSpec((hid, npad), lambda i: (0, 0)),
            pl.BlockSpec((1, npad), lambda i: (0, 0)),
        ],
        out_specs=pl.BlockSpec((nblk, npad), lambda i: (i, 0)),
        compiler_params=pltpu.CompilerParams(
            dimension_semantics=("parallel",),
            vmem_limit_bytes=64 * 1024 * 1024),
    )(feat2, w1p, b1p, whp, bhp)

    y2 = y_all.reshape(n, wh.shape[1])
    outs, off = [], 0
    for _ in range(10):
        outs.append(y2[:, off:off + 10])
        off += 10
    return outs
